# Initial kernel scaffold; baseline (speedup 1.0000x reference)
import jax
import jax.numpy as jnp
from jax.experimental import pallas as pl


def kernel(x, edge_index, params):
    raise NotImplementedError("write your pallas kernel here")


# speedup vs baseline: 1.0300x; 1.0300x over previous
import jax
import jax.numpy as jnp
from jax.experimental import pallas as pl

HEADS = 8
NUM_LAYERS = 2


def _mm_bias(x, W, b, block_rows=1000):
    M, K = x.shape
    Nc = W.shape[1]

    def kern(x_ref, w_ref, b_ref, o_ref):
        o_ref[...] = (
            jnp.dot(x_ref[...], w_ref[...], preferred_element_type=jnp.float32)
            + b_ref[...]
        )

    return pl.pallas_call(
        kern,
        grid=(M // block_rows,),
        in_specs=[
            pl.BlockSpec((block_rows, K), lambda i: (i, 0)),
            pl.BlockSpec((K, Nc), lambda i: (0, 0)),
            pl.BlockSpec((1, Nc), lambda i: (0, 0)),
        ],
        out_specs=pl.BlockSpec((block_rows, Nc), lambda i: (i, 0)),
        out_shape=jax.ShapeDtypeStruct((M, Nc), jnp.float32),
    )(x, W, b.reshape(1, -1))


def _block_tail(x, skip, attn, g1, b1v, W1, bb1, W2, bb2, g2, b2v,
                block_rows=1000):
    M, C = x.shape
    F = W1.shape[1]

    def kern(x_ref, s_ref, a_ref, g1_ref, b1_ref, w1_ref, bb1_ref,
             w2_ref, bb2_ref, g2_ref, b2_ref, o_ref):
        x1 = x_ref[...] + s_ref[...] + a_ref[...]
        m = jnp.mean(x1, axis=-1, keepdims=True)
        v = jnp.mean((x1 - m) ** 2, axis=-1, keepdims=True)
        y = (x1 - m) / jnp.sqrt(v + 1e-5) * g1_ref[...] + b1_ref[...]
        h = jnp.maximum(
            jnp.dot(y, w1_ref[...], preferred_element_type=jnp.float32)
            + bb1_ref[...], 0.0)
        h = (jnp.dot(h, w2_ref[...], preferred_element_type=jnp.float32)
             + bb2_ref[...])
        z = y + h
        m2 = jnp.mean(z, axis=-1, keepdims=True)
        v2 = jnp.mean((z - m2) ** 2, axis=-1, keepdims=True)
        o_ref[...] = (z - m2) / jnp.sqrt(v2 + 1e-5) * g2_ref[...] + b2_ref[...]

    row = lambda i: (i, 0)
    fixed = lambda i: (0, 0)
    return pl.pallas_call(
        kern,
        grid=(M // block_rows,),
        in_specs=[
            pl.BlockSpec((block_rows, C), row),
            pl.BlockSpec((block_rows, C), row),
            pl.BlockSpec((block_rows, C), row),
            pl.BlockSpec((1, C), fixed),
            pl.BlockSpec((1, C), fixed),
            pl.BlockSpec((C, F), fixed),
            pl.BlockSpec((1, F), fixed),
            pl.BlockSpec((F, C), fixed),
            pl.BlockSpec((1, C), fixed),
            pl.BlockSpec((1, C), fixed),
            pl.BlockSpec((1, C), fixed),
        ],
        out_specs=pl.BlockSpec((block_rows, C), row),
        out_shape=jax.ShapeDtypeStruct((M, C), jnp.float32),
    )(x, skip, attn, g1.reshape(1, -1), b1v.reshape(1, -1), W1,
      bb1.reshape(1, -1), W2, bb2.reshape(1, -1), g2.reshape(1, -1),
      b2v.reshape(1, -1))


def _edge_attention(q, k, v, src, dst, n):
    h = HEADS
    c = q.shape[1]
    d = c // h
    qe = q.reshape(n, h, d)
    ke = k.reshape(n, h, d)
    alpha = jnp.sum(qe[dst] * ke[src], axis=-1) / jnp.sqrt(jnp.float32(d))
    amax = jax.ops.segment_max(alpha, dst, num_segments=n)
    amax = jnp.where(jnp.isfinite(amax), amax, 0.0)
    ex = jnp.exp(alpha - amax[dst])
    denom = jax.ops.segment_sum(ex, dst, num_segments=n)
    msg = v.reshape(n, h, d)[src] * ex[..., None]
    numer = jax.ops.segment_sum(msg, dst, num_segments=n).reshape(n, c)
    return numer, denom


def kernel(x, edge_index, params):
    src = edge_index[0].astype(jnp.int32)
    dst = edge_index[1].astype(jnp.int32)
    n, c = x.shape

    x = _mm_bias(x, params['in_W'], params['in_b'])
    for l in range(NUM_LAYERS):
        p = params['blocks'][l]
        Wqkvs = jnp.concatenate([p['Wq'], p['Wk'], p['Wv'], p['Wskip']], axis=1)
        bqkvs = jnp.concatenate([p['bq'], p['bk'], p['bv'], p['bskip']])
        qkvs = _mm_bias(x, Wqkvs, bqkvs)
        q = qkvs[:, :c]
        k = qkvs[:, c:2 * c]
        v = qkvs[:, 2 * c:3 * c]
        skip = qkvs[:, 3 * c:]
        numer, denom = _edge_attention(q, k, v, src, dst, n)
        attn = (numer.reshape(n, HEADS, c // HEADS)
                / (denom[:, :, None] + 1e-16)).reshape(n, c)
        x = _block_tail(x, skip, attn, p['ln1_g'], p['ln1_b'], p['W1'],
                        p['b1'], p['W2'], p['b2'], p['ln2_g'], p['ln2_b'])
    x = _mm_bias(x, params['out_W'], params['out_b'])
    return x


# speedup vs baseline: 1.0812x; 1.0497x over previous
import functools

import jax
import jax.numpy as jnp
from jax import lax
from jax.experimental import pallas as pl
from jax.experimental.pallas import tpu as pltpu
from jax.experimental.pallas import tpu_sc as plsc

HEADS = 8
NUM_LAYERS = 2
N = 10000
C = 256
E = 160000
E_PAD = 163840
GA = 32
PW_A = E_PAD // 32
NCH_A = PW_A // GA
GB = 64
PW_B = E_PAD // 16
NCH_B = PW_B // GB
RPS = 624
_SCALE = 1.0 / (32.0 ** 0.5)

_MESH = plsc.VectorSubcoreMesh(core_axis_name="c", subcore_axis_name="s")


def _mm_bias(x, W, b, block_rows=1000):
    M, K = x.shape
    Nc = W.shape[1]

    def kern(x_ref, w_ref, b_ref, o_ref):
        o_ref[...] = (
            jnp.dot(x_ref[...], w_ref[...], preferred_element_type=jnp.float32)
            + b_ref[...]
        )

    return pl.pallas_call(
        kern,
        grid=(M // block_rows,),
        in_specs=[
            pl.BlockSpec((block_rows, K), lambda i: (i, 0)),
            pl.BlockSpec((K, Nc), lambda i: (0, 0)),
            pl.BlockSpec((1, Nc), lambda i: (0, 0)),
        ],
        out_specs=pl.BlockSpec((block_rows, Nc), lambda i: (i, 0)),
        out_shape=jax.ShapeDtypeStruct((M, Nc), jnp.float32),
    )(x, W, b.reshape(1, -1))


def _block_tail(x, skip, numer, denom, g1, b1v, W1, bb1, W2, bb2, g2, b2v,
                block_rows=1000):
    M, Cc = x.shape
    F = W1.shape[1]

    def kern(x_ref, s_ref, nu_ref, de_ref, g1_ref, b1_ref, w1_ref, bb1_ref,
             w2_ref, bb2_ref, g2_ref, b2_ref, o_ref):
        attn = nu_ref[...] / (de_ref[...] + 1e-16)
        x1 = x_ref[...] + s_ref[...] + attn
        m = jnp.mean(x1, axis=-1, keepdims=True)
        v = jnp.mean((x1 - m) ** 2, axis=-1, keepdims=True)
        y = (x1 - m) / jnp.sqrt(v + 1e-5) * g1_ref[...] + b1_ref[...]
        h = jnp.maximum(
            jnp.dot(y, w1_ref[...], preferred_element_type=jnp.float32)
            + bb1_ref[...], 0.0)
        h = (jnp.dot(h, w2_ref[...], preferred_element_type=jnp.float32)
             + bb2_ref[...])
        z = y + h
        m2 = jnp.mean(z, axis=-1, keepdims=True)
        v2 = jnp.mean((z - m2) ** 2, axis=-1, keepdims=True)
        o_ref[...] = (z - m2) / jnp.sqrt(v2 + 1e-5) * g2_ref[...] + b2_ref[...]

    row = lambda i: (i, 0)
    fixed = lambda i: (0, 0)
    return pl.pallas_call(
        kern,
        grid=(M // block_rows,),
        in_specs=[
            pl.BlockSpec((block_rows, Cc), row),
            pl.BlockSpec((block_rows, Cc), row),
            pl.BlockSpec((block_rows, Cc), row),
            pl.BlockSpec((block_rows, Cc), row),
            pl.BlockSpec((1, Cc), fixed),
            pl.BlockSpec((1, Cc), fixed),
            pl.BlockSpec((Cc, F), fixed),
            pl.BlockSpec((1, F), fixed),
            pl.BlockSpec((F, Cc), fixed),
            pl.BlockSpec((1, Cc), fixed),
            pl.BlockSpec((1, Cc), fixed),
            pl.BlockSpec((1, Cc), fixed),
        ],
        out_specs=pl.BlockSpec((block_rows, Cc), row),
        out_shape=jax.ShapeDtypeStruct((M, Cc), jnp.float32),
    )(x, skip, numer, denom, g1.reshape(1, -1), b1v.reshape(1, -1), W1,
      bb1.reshape(1, -1), W2, bb2.reshape(1, -1), g2.reshape(1, -1),
      b2v.reshape(1, -1))


@functools.partial(
    pl.kernel,
    out_type=[
        jax.ShapeDtypeStruct((E_PAD * 8,), jnp.float32),
        jax.ShapeDtypeStruct((512,), jnp.float32),
    ],
    scratch_types=[
        pltpu.VMEM((GA,), jnp.int32),
        pltpu.VMEM((GA,), jnp.int32),
        pltpu.VMEM((GA, C), jnp.float32),
        pltpu.VMEM((GA, C), jnp.float32),
        pltpu.VMEM((GA * 8,), jnp.float32),
        pltpu.VMEM((16,), jnp.float32),
        pltpu.SemaphoreType.DMA,
        pltpu.SemaphoreType.DMA,
    ],
    compiler_params=pltpu.CompilerParams(needs_layout_passes=False),
    mesh=_MESH,
)
def _edge_alpha(q_hbm, k_hbm, gdst_hbm, src_hbm, alpha_hbm, tmax_hbm,
                di_v, si_v, qbuf, kbuf, abuf, mbuf, sem1, sem2):
    c = lax.axis_index("c")
    s = lax.axis_index("s")
    wid = s * 2 + c
    base = wid * PW_A
    iota = lax.iota(jnp.int32, 16)
    iota8 = iota * 8

    def chunk(i, mx):
        eoff = base + i * GA
        pltpu.sync_copy(gdst_hbm.at[pl.ds(eoff, GA)], di_v)
        pltpu.sync_copy(src_hbm.at[pl.ds(eoff, GA)], si_v)
        cp1 = pltpu.async_copy(q_hbm.at[di_v], qbuf, sem1)
        cp2 = pltpu.async_copy(k_hbm.at[si_v], kbuf, sem2)
        cp1.wait()
        cp2.wait()
        for g in range(GA // 16):
            rows = iota + (g * 16)
            for h in range(HEADS):
                @plsc.parallel_loop(h * 32, h * 32 + 32, step=1, unroll=8,
                                    carry=jnp.zeros((16,), jnp.float32))
                def facc(f, acc):
                    colv = jnp.zeros((16,), jnp.int32) + f
                    qv = plsc.load_gather(qbuf, [rows, colv])
                    kv = plsc.load_gather(kbuf, [rows, colv])
                    return acc + qv * kv

                ah = facc * _SCALE
                mx = jnp.maximum(mx, ah)
                plsc.store_scatter(abuf, [iota8 + (g * 128 + h)], ah)
        pltpu.sync_copy(abuf, alpha_hbm.at[pl.ds(eoff * 8, GA * 8)])
        return mx

    mx = lax.fori_loop(0, NCH_A, chunk, jnp.full((16,), -jnp.inf, jnp.float32))
    mbuf[...] = mx
    pltpu.sync_copy(mbuf, tmax_hbm.at[pl.ds(wid * 16, 16)])


@functools.partial(
    pl.kernel,
    out_type=[
        jax.ShapeDtypeStruct((2 * N, 128), jnp.float32),
        jax.ShapeDtypeStruct((N, 16), jnp.float32),
    ],
    scratch_types=[
        pltpu.VMEM((GB,), jnp.int32),
        pltpu.VMEM((GB,), jnp.int32),
        pltpu.VMEM((GB, 128), jnp.float32),
        pltpu.VMEM((GB, 16), jnp.float32),
        pltpu.VMEM((GB * 8,), jnp.float32),
        pltpu.VMEM((512,), jnp.float32),
        pltpu.VMEM((GB, 16), jnp.float32),
        pltpu.VMEM_SHARED((N + 16, 128), jnp.float32),
        pltpu.VMEM_SHARED((N + 16, 16), jnp.float32),
        pltpu.SemaphoreType.DMA,
    ],
    compiler_params=pltpu.CompilerParams(needs_layout_passes=False),
    mesh=_MESH,
)
def _edge_aggregate(v_hbm, src_hbm, sdst_hbm, alpha_hbm, tmax_hbm,
                    numer_hbm, den_hbm, si_v, di_v, vbuf, exbuf, abuf,
                    tbuf, zbuf, acc_sh, den_sh, sem):
    c = lax.axis_index("c")
    s = lax.axis_index("s")
    cN = c * N
    iota = lax.iota(jnp.int32, 16)
    prow = iota // 8
    pcol = iota - prow * 8
    hcol = [jnp.full((16,), jj, jnp.int32) + c * 4 for jj in range(4)]
    zv = jnp.zeros((16,), jnp.float32)

    def zrow(i, t):
        for j in range(8):
            vbuf[i, pl.ds(j * 16, 16)] = zv
        zbuf[i, pl.ds(0, 16)] = zv
        return t

    lax.fori_loop(0, GB, zrow, 0)
    rbase = s * RPS
    for r0 in range(0, RPS, GB):
        nr = min(GB, RPS - r0)
        pltpu.sync_copy(vbuf.at[pl.ds(0, nr)], acc_sh.at[pl.ds(rbase + r0, nr)])

        @pl.when(c == 0)
        def _():
            pltpu.sync_copy(zbuf.at[pl.ds(0, nr)],
                            den_sh.at[pl.ds(rbase + r0, nr)])

    @pl.when(s == 15)
    def _():
        pltpu.sync_copy(vbuf.at[pl.ds(0, 32)], acc_sh.at[pl.ds(15 * RPS + RPS, 32)])

        @pl.when(c == 0)
        def _():
            pltpu.sync_copy(zbuf.at[pl.ds(0, 32)],
                            den_sh.at[pl.ds(15 * RPS + RPS, 32)])

    plsc.subcore_barrier()

    pltpu.sync_copy(tmax_hbm, tbuf)
    mv = tbuf[pl.ds(0, 16)]
    for i in range(1, 32):
        mv = jnp.maximum(mv, tbuf[pl.ds(i * 16, 16)])
    K = jnp.max(mv)

    ebase = s * PW_B

    def chunk(i, t):
        eoff = ebase + i * GB
        pltpu.sync_copy(src_hbm.at[pl.ds(eoff, GB)], si_v)
        pltpu.sync_copy(sdst_hbm.at[pl.ds(eoff, GB)], di_v)
        for j in range(GB // 16):
            si_v[pl.ds(j * 16, 16)] = si_v[pl.ds(j * 16, 16)] + cN
        cp = pltpu.async_copy(v_hbm.at[si_v], vbuf, sem)
        pltpu.sync_copy(alpha_hbm.at[pl.ds(eoff * 8, GB * 8)], abuf)
        for m in range(GB * 8 // 16):
            av = abuf[pl.ds(m * 16, 16)]
            ev = jnp.exp(av - K)
            plsc.store_scatter(exbuf, [prow + 2 * m, pcol], ev)
        cp.wait()
        @plsc.parallel_loop(0, GB, step=1, unroll=2)
        def estep(e):
            rowv = jnp.zeros((16,), jnp.int32) + e
            for jj in range(4):
                exs = plsc.load_gather(exbuf, [rowv, hcol[jj]])
                for j2 in (2 * jj, 2 * jj + 1):
                    vv = vbuf[e, pl.ds(j2 * 16, 16)]
                    vbuf[e, pl.ds(j2 * 16, 16)] = vv * exs
        pltpu.sync_copy(vbuf, acc_sh.at[di_v], add=True)

        @pl.when(c == 0)
        def _():
            pltpu.sync_copy(exbuf, den_sh.at[di_v], add=True)

        return t

    lax.fori_loop(0, NCH_B, chunk, 0)
    plsc.subcore_barrier()
    pltpu.sync_copy(acc_sh.at[pl.ds(rbase, RPS)],
                    numer_hbm.at[pl.ds(cN + rbase, RPS)])

    @pl.when(c == 0)
    def _():
        pltpu.sync_copy(den_sh.at[pl.ds(rbase, RPS)],
                        den_hbm.at[pl.ds(rbase, RPS)])

    @pl.when(s == 15)
    def _():
        pltpu.sync_copy(acc_sh.at[pl.ds(16 * RPS, 16)],
                        numer_hbm.at[pl.ds(cN + 16 * RPS, 16)])

        @pl.when(c == 0)
        def _():
            pltpu.sync_copy(den_sh.at[pl.ds(16 * RPS, 16)],
                            den_hbm.at[pl.ds(16 * RPS, 16)])


def kernel(x, edge_index, params):
    src = edge_index[0].astype(jnp.int32)
    dst = edge_index[1].astype(jnp.int32)
    n, cdim = x.shape
    pad = E_PAD - E
    srcp = jnp.pad(src, (0, pad))
    gdst = jnp.pad(dst, (0, pad))
    sdst = jnp.pad(dst, (0, pad), constant_values=N)

    x = _mm_bias(x, params['in_W'], params['in_b'])
    for l in range(NUM_LAYERS):
        p = params['blocks'][l]
        Wqkvs = jnp.concatenate([p['Wq'], p['Wk'], p['Wv'], p['Wskip']],
                                axis=1)
        bqkvs = jnp.concatenate([p['bq'], p['bk'], p['bv'], p['bskip']])
        qkvs = _mm_bias(x, Wqkvs, bqkvs)
        q = qkvs[:, :cdim]
        k = qkvs[:, cdim:2 * cdim]
        v = qkvs[:, 2 * cdim:3 * cdim]
        skip = qkvs[:, 3 * cdim:]
        v2 = v.reshape(n, 2, 128).transpose(1, 0, 2).reshape(2 * n, 128)
        alpha, tmax = _edge_alpha(q, k, gdst, srcp)
        K = jnp.max(tmax)
        ex = jnp.exp(alpha.reshape(E_PAD, 8)[:E] - K)
        den = jax.ops.segment_sum(ex, dst, num_segments=n)
        msg = v.reshape(n, HEADS, 32)[src] * ex[..., None]
        numer = jax.ops.segment_sum(msg, dst, num_segments=n).reshape(n, cdim)
        denom = jnp.repeat(den, cdim // HEADS, axis=1)
        x = _block_tail(x, skip, numer, denom, p['ln1_g'], p['ln1_b'],
                        p['W1'], p['b1'], p['W2'], p['b2'], p['ln2_g'],
                        p['ln2_b'])
    x = _mm_bias(x, params['out_W'], params['out_b'])
    return x
